# baseline (speedup 1.0000x reference)
import functools

import jax
import jax.numpy as jnp
from jax import lax
from jax.experimental import pallas as pl
from jax.experimental.pallas import tpu as pltpu
from jax.experimental.pallas import tpu_sc as plsc

N = 2048
K = 32
H = 128
R = N * K
FF = 4 * H
SCALE = 30.0

NODE_BLK = 256
EDGE_BLK = NODE_BLK * K
NBLK = N // NODE_BLK

CHUNK = 128
NWORKERS = 32
CPW = R // (CHUNK * NWORKERS)

_SQRT_HALF = 0.7071067811865476


def _gelu(x):
    return 0.5 * x * (1.0 + lax.erf(x * _SQRT_HALF))


def _dot16(a, b):
    return jnp.dot(a.astype(jnp.bfloat16), b.astype(jnp.bfloat16),
                   preferred_element_type=jnp.float32)


def _ln(x, g, b):
    mu = jnp.mean(x, axis=-1, keepdims=True)
    xc = x - mu
    var = jnp.mean(xc * xc, axis=-1, keepdims=True)
    return xc / jnp.sqrt(var + 1e-5) * g + b


def _prep_body(hv_ref, w_ref, q_ref):
    q_ref[...] = jnp.dot(hv_ref[...], w_ref[...])


def _tc_prep(hv, w1c):
    return pl.pallas_call(
        _prep_body,
        out_shape=jax.ShapeDtypeStruct((N, H), jnp.float32),
    )(hv, w1c)


def _sc_gather(table, idx2d):
    @functools.partial(
        pl.kernel,
        mesh=plsc.VectorSubcoreMesh(core_axis_name="c", subcore_axis_name="s"),
        out_type=jax.ShapeDtypeStruct((R, H), jnp.float32),
        scratch_types=[
            pltpu.VMEM((CHUNK,), jnp.int32),
            pltpu.VMEM((CHUNK, H), jnp.float32),
            pltpu.SemaphoreType.DMA,
        ],
    )
    def k(table_hbm, idx_hbm, out_hbm, idx_v, rows_v, sem):
        wid = lax.axis_index("s") * 2 + lax.axis_index("c")
        base = wid * CPW

        def body(t, carry):
            c = base + t
            pltpu.sync_copy(idx_hbm.at[c], idx_v)
            pltpu.async_copy(table_hbm.at[idx_v], rows_v, sem).wait()
            pltpu.sync_copy(rows_v, out_hbm.at[pl.ds(c * CHUNK, CHUNK)])
            return carry

        lax.fori_loop(0, CPW, body, 0)

    return k(table, idx2d)


def _main_body(hv_ref, he_ref, g1_ref, ma_ref, mv_ref,
               w1a_ref, b1_ref, w1b_ref, w2_ref, b2_ref, w3_ref, b3_ref,
               wdin_ref, bdin_ref, wdout_ref, bdout_ref,
               ln1g_ref, ln1b_ref, ln2g_ref, ln2b_ref,
               w11a_ref, b11_ref, w11c_ref,
               hv2_ref, q2_ref, pre2_ref):
    hv = hv_ref[...]
    pre1 = _dot16(hv, w1a_ref[...]) + b1_ref[...]
    x = _dot16(he_ref[...], w1b_ref[...]) + g1_ref[...]
    x = (x.reshape(NODE_BLK, K, H) + pre1[:, None, :]).reshape(EDGE_BLK, H)
    x = _gelu(x)
    x = _gelu(_dot16(x, w2_ref[...]) + b2_ref[...])
    m = _dot16(x, w3_ref[...]) + b3_ref[...]
    m = m * ma_ref[...]
    dh = jnp.sum(m.reshape(NODE_BLK, K, H), axis=1) * (1.0 / SCALE)
    hv2 = _ln(hv + dh, ln1g_ref[...], ln1b_ref[...])
    ffn = _dot16(_gelu(_dot16(hv2, wdin_ref[...]) + bdin_ref[...]),
                 wdout_ref[...]) + bdout_ref[...]
    hv2 = _ln(hv2 + ffn, ln2g_ref[...], ln2b_ref[...])
    hv2 = hv2 * mv_ref[...]
    hv2_ref[...] = hv2
    q2_ref[...] = jnp.dot(hv2, w11c_ref[...])
    pre2_ref[...] = _dot16(hv2, w11a_ref[...]) + b11_ref[...]


def _tc_main(hv, he, g1, ma, mv, w1a, b1r, w1b, w2, b2r, w3, b3r,
             wdin, bdinr, wdout, bdoutr, ln1g, ln1b, ln2g, ln2b,
             w11a, b11r, w11c):
    node_spec = pl.BlockSpec((NODE_BLK, H), lambda i: (i, 0))
    edge_spec = pl.BlockSpec((EDGE_BLK, H), lambda i: (i, 0))

    def full(a):
        return pl.BlockSpec(a.shape, lambda i: (0,) * a.ndim)

    in_specs = [
        node_spec, edge_spec, edge_spec,
        pl.BlockSpec((EDGE_BLK, 1), lambda i: (i, 0)),
        pl.BlockSpec((NODE_BLK, 1), lambda i: (i, 0)),
        full(w1a), full(b1r), full(w1b), full(w2), full(b2r), full(w3),
        full(b3r), full(wdin), full(bdinr), full(wdout), full(bdoutr),
        full(ln1g), full(ln1b), full(ln2g), full(ln2b),
        full(w11a), full(b11r), full(w11c),
    ]
    out_specs = [node_spec, node_spec, node_spec]
    out_shape = [jax.ShapeDtypeStruct((N, H), jnp.float32)] * 3
    return pl.pallas_call(
        _main_body,
        grid=(NBLK,),
        in_specs=in_specs,
        out_specs=out_specs,
        out_shape=out_shape,
        compiler_params=pltpu.CompilerParams(
            dimension_semantics=("arbitrary",)),
    )(hv, he, g1, ma, mv, w1a, b1r, w1b, w2, b2r, w3, b3r, wdin, bdinr,
      wdout, bdoutr, ln1g, ln1b, ln2g, ln2b, w11a, b11r, w11c)


def _edge_body(he_ref, g2_ref, pre2_ref, w11b_ref, w12_ref, b12_ref,
               w13_ref, b13_ref, ln3g_ref, ln3b_ref, he2_ref):
    he = he_ref[...]
    x = _dot16(he, w11b_ref[...]) + g2_ref[...]
    x = (x.reshape(NODE_BLK, K, H) + pre2_ref[...][:, None, :]).reshape(EDGE_BLK, H)
    x = _gelu(x)
    x = _gelu(_dot16(x, w12_ref[...]) + b12_ref[...])
    m = _dot16(x, w13_ref[...]) + b13_ref[...]
    he2_ref[...] = _ln(he + m, ln3g_ref[...], ln3b_ref[...])


def _tc_edge(he, g2, pre2, w11b, w12, b12r, w13, b13r, ln3g, ln3b):
    node_spec = pl.BlockSpec((NODE_BLK, H), lambda i: (i, 0))
    edge_spec = pl.BlockSpec((EDGE_BLK, H), lambda i: (i, 0))

    def full(a):
        return pl.BlockSpec(a.shape, lambda i: (0,) * a.ndim)

    in_specs = [
        edge_spec, edge_spec, node_spec,
        full(w11b), full(w12), full(b12r), full(w13), full(b13r),
        full(ln3g), full(ln3b),
    ]
    return pl.pallas_call(
        _edge_body,
        grid=(NBLK,),
        in_specs=in_specs,
        out_specs=edge_spec,
        out_shape=jax.ShapeDtypeStruct((R, H), jnp.float32),
        compiler_params=pltpu.CompilerParams(
            dimension_semantics=("arbitrary",)),
    )(he, g2, pre2, w11b, w12, b12r, w13, b13r, ln3g, ln3b)


def kernel(h_V, h_E, mask_V, mask_attend, W1, b1, W2, b2, W3, b3,
           Wd_in, bd_in, Wd_out, bd_out, W11, b11, W12, b12, W13, b13,
           ln1_g, ln1_b, ln2_g, ln2_b, ln3_g, ln3_b, E_idx):
    hv = h_V.reshape(N, H)
    he = h_E.reshape(R, H)
    ma = mask_attend.reshape(R, 1)
    mv = mask_V.reshape(N, 1)
    idx2d = E_idx.reshape(R // CHUNK, CHUNK).astype(jnp.int32)

    w1a, w1b, w1c = W1[:H], W1[H:2 * H], W1[2 * H:]
    w11a, w11b, w11c = W11[:H], W11[H:2 * H], W11[2 * H:]
    row = lambda v: v.reshape(1, -1)

    q1 = _tc_prep(hv, w1c)
    g1 = _sc_gather(q1, idx2d)
    hv2, q2, pre2 = _tc_main(
        hv, he, g1, ma, mv, w1a, row(b1), w1b, W2, row(b2), W3, row(b3),
        Wd_in, row(bd_in), Wd_out, row(bd_out),
        row(ln1_g), row(ln1_b), row(ln2_g), row(ln2_b),
        w11a, row(b11), w11c)
    g2 = _sc_gather(q2, idx2d)
    he2 = _tc_edge(he, g2, pre2, w11b, W12, row(b12), W13, row(b13),
                   row(ln3_g), row(ln3_b))
    return (hv2.reshape(1, N, H), he2.reshape(1, N, K, H))


# speedup vs baseline: 1.0829x; 1.0829x over previous
import functools

import jax
import jax.numpy as jnp
from jax import lax
from jax.experimental import pallas as pl
from jax.experimental.pallas import tpu as pltpu
from jax.experimental.pallas import tpu_sc as plsc

N = 2048
K = 32
H = 128
R = N * K
FF = 4 * H
SCALE = 30.0

NODE_BLK = 256
EDGE_BLK = NODE_BLK * K
NBLK = N // NODE_BLK

CHUNK = 128
NWORKERS = 32
CPW = R // (CHUNK * NWORKERS)

_SQRT_HALF = 0.7071067811865476


def _gelu(x):
    return 0.5 * x * (1.0 + lax.erf(x * _SQRT_HALF))


def _dot16(a, b):
    return jnp.dot(a.astype(jnp.bfloat16), b.astype(jnp.bfloat16),
                   preferred_element_type=jnp.float32)


def _ln(x, g, b):
    mu = jnp.mean(x, axis=-1, keepdims=True)
    xc = x - mu
    var = jnp.mean(xc * xc, axis=-1, keepdims=True)
    return xc / jnp.sqrt(var + 1e-5) * g + b


def _prep_body(hv_ref, w_ref, q_ref):
    q_ref[...] = jnp.dot(hv_ref[...], w_ref[...])


def _tc_prep(hv, w1c):
    return pl.pallas_call(
        _prep_body,
        out_shape=jax.ShapeDtypeStruct((N, H), jnp.float32),
    )(hv, w1c)


def _sc_gather(table, idx2d):
    NBUF = 3

    @functools.partial(
        pl.kernel,
        mesh=plsc.VectorSubcoreMesh(core_axis_name="c", subcore_axis_name="s"),
        out_type=jax.ShapeDtypeStruct((R, H), jnp.float32),
        scratch_types=[
            pltpu.VMEM((CPW, CHUNK), jnp.int32),
            pltpu.VMEM((NBUF, CHUNK, H), jnp.float32),
            pltpu.SemaphoreType.DMA,
            pltpu.SemaphoreType.DMA((NBUF,)),
            pltpu.SemaphoreType.DMA((NBUF,)),
        ],
    )
    def k(table_hbm, idx_hbm, out_hbm, idx_v, rows_v, sem_i, sem_g, sem_w):
        wid = lax.axis_index("s") * 2 + lax.axis_index("c")
        base = wid * CPW
        pltpu.async_copy(idx_hbm.at[pl.ds(base, CPW)], idx_v, sem_i).wait()

        def start_gather(t):
            return pltpu.async_copy(table_hbm.at[idx_v.at[t]],
                                    rows_v.at[t % NBUF], sem_g.at[t % NBUF])

        gh = {t: start_gather(t) for t in range(min(NBUF, CPW))}
        wh = {}
        for t in range(CPW):
            b = t % NBUF
            gh[t].wait()
            wh[t] = pltpu.async_copy(
                rows_v.at[b], out_hbm.at[pl.ds((base + t) * CHUNK, CHUNK)],
                sem_w.at[b])
            if t + NBUF < CPW:
                wh[t].wait()
                gh[t + NBUF] = start_gather(t + NBUF)
        for t in range(max(0, CPW - NBUF), CPW):
            wh[t].wait()

    return k(table, idx2d)


def _main_body(hv_ref, he_ref, g1_ref, ma_ref, mv_ref,
               w1a_ref, b1_ref, w1b_ref, w2_ref, b2_ref, w3_ref, b3_ref,
               wdin_ref, bdin_ref, wdout_ref, bdout_ref,
               ln1g_ref, ln1b_ref, ln2g_ref, ln2b_ref,
               w11a_ref, b11_ref, w11c_ref,
               hv2_ref, q2_ref, pre2_ref):
    hv = hv_ref[...]
    pre1 = _dot16(hv, w1a_ref[...]) + b1_ref[...]
    x = _dot16(he_ref[...], w1b_ref[...]) + g1_ref[...]
    x = (x.reshape(NODE_BLK, K, H) + pre1[:, None, :]).reshape(EDGE_BLK, H)
    x = _gelu(x)
    x = _gelu(_dot16(x, w2_ref[...]) + b2_ref[...])
    m = _dot16(x, w3_ref[...]) + b3_ref[...]
    m = m * ma_ref[...]
    dh = jnp.sum(m.reshape(NODE_BLK, K, H), axis=1) * (1.0 / SCALE)
    hv2 = _ln(hv + dh, ln1g_ref[...], ln1b_ref[...])
    ffn = _dot16(_gelu(_dot16(hv2, wdin_ref[...]) + bdin_ref[...]),
                 wdout_ref[...]) + bdout_ref[...]
    hv2 = _ln(hv2 + ffn, ln2g_ref[...], ln2b_ref[...])
    hv2 = hv2 * mv_ref[...]
    hv2_ref[...] = hv2
    q2_ref[...] = jnp.dot(hv2, w11c_ref[...])
    pre2_ref[...] = _dot16(hv2, w11a_ref[...]) + b11_ref[...]


def _tc_main(hv, he, g1, ma, mv, w1a, b1r, w1b, w2, b2r, w3, b3r,
             wdin, bdinr, wdout, bdoutr, ln1g, ln1b, ln2g, ln2b,
             w11a, b11r, w11c):
    node_spec = pl.BlockSpec((NODE_BLK, H), lambda i: (i, 0))
    edge_spec = pl.BlockSpec((EDGE_BLK, H), lambda i: (i, 0))

    def full(a):
        return pl.BlockSpec(a.shape, lambda i: (0,) * a.ndim)

    in_specs = [
        node_spec, edge_spec, edge_spec,
        pl.BlockSpec((EDGE_BLK, 1), lambda i: (i, 0)),
        pl.BlockSpec((NODE_BLK, 1), lambda i: (i, 0)),
        full(w1a), full(b1r), full(w1b), full(w2), full(b2r), full(w3),
        full(b3r), full(wdin), full(bdinr), full(wdout), full(bdoutr),
        full(ln1g), full(ln1b), full(ln2g), full(ln2b),
        full(w11a), full(b11r), full(w11c),
    ]
    out_specs = [node_spec, node_spec, node_spec]
    out_shape = [jax.ShapeDtypeStruct((N, H), jnp.float32)] * 3
    return pl.pallas_call(
        _main_body,
        grid=(NBLK,),
        in_specs=in_specs,
        out_specs=out_specs,
        out_shape=out_shape,
        compiler_params=pltpu.CompilerParams(
            dimension_semantics=("arbitrary",)),
    )(hv, he, g1, ma, mv, w1a, b1r, w1b, w2, b2r, w3, b3r, wdin, bdinr,
      wdout, bdoutr, ln1g, ln1b, ln2g, ln2b, w11a, b11r, w11c)


def _edge_body(he_ref, g2_ref, pre2_ref, w11b_ref, w12_ref, b12_ref,
               w13_ref, b13_ref, ln3g_ref, ln3b_ref, he2_ref):
    he = he_ref[...]
    x = _dot16(he, w11b_ref[...]) + g2_ref[...]
    x = (x.reshape(NODE_BLK, K, H) + pre2_ref[...][:, None, :]).reshape(EDGE_BLK, H)
    x = _gelu(x)
    x = _gelu(_dot16(x, w12_ref[...]) + b12_ref[...])
    m = _dot16(x, w13_ref[...]) + b13_ref[...]
    he2_ref[...] = _ln(he + m, ln3g_ref[...], ln3b_ref[...])


def _tc_edge(he, g2, pre2, w11b, w12, b12r, w13, b13r, ln3g, ln3b):
    node_spec = pl.BlockSpec((NODE_BLK, H), lambda i: (i, 0))
    edge_spec = pl.BlockSpec((EDGE_BLK, H), lambda i: (i, 0))

    def full(a):
        return pl.BlockSpec(a.shape, lambda i: (0,) * a.ndim)

    in_specs = [
        edge_spec, edge_spec, node_spec,
        full(w11b), full(w12), full(b12r), full(w13), full(b13r),
        full(ln3g), full(ln3b),
    ]
    return pl.pallas_call(
        _edge_body,
        grid=(NBLK,),
        in_specs=in_specs,
        out_specs=edge_spec,
        out_shape=jax.ShapeDtypeStruct((R, H), jnp.float32),
        compiler_params=pltpu.CompilerParams(
            dimension_semantics=("arbitrary",)),
    )(he, g2, pre2, w11b, w12, b12r, w13, b13r, ln3g, ln3b)


def kernel(h_V, h_E, mask_V, mask_attend, W1, b1, W2, b2, W3, b3,
           Wd_in, bd_in, Wd_out, bd_out, W11, b11, W12, b12, W13, b13,
           ln1_g, ln1_b, ln2_g, ln2_b, ln3_g, ln3_b, E_idx):
    hv = h_V.reshape(N, H)
    he = h_E.reshape(R, H)
    ma = mask_attend.reshape(R, 1)
    mv = mask_V.reshape(N, 1)
    idx2d = E_idx.reshape(R // CHUNK, CHUNK).astype(jnp.int32)

    w1a, w1b, w1c = W1[:H], W1[H:2 * H], W1[2 * H:]
    w11a, w11b, w11c = W11[:H], W11[H:2 * H], W11[2 * H:]
    row = lambda v: v.reshape(1, -1)

    q1 = _tc_prep(hv, w1c)
    g1 = _sc_gather(q1, idx2d)
    hv2, q2, pre2 = _tc_main(
        hv, he, g1, ma, mv, w1a, row(b1), w1b, W2, row(b2), W3, row(b3),
        Wd_in, row(bd_in), Wd_out, row(bd_out),
        row(ln1_g), row(ln1_b), row(ln2_g), row(ln2_b),
        w11a, row(b11), w11c)
    g2 = _sc_gather(q2, idx2d)
    he2 = _tc_edge(he, g2, pre2, w11b, W12, row(b12), W13, row(b13),
                   row(ln3_g), row(ln3_b))
    return (hv2.reshape(1, N, H), he2.reshape(1, N, K, H))
